# baseline (speedup 1.0000x reference)
import functools

import jax
import jax.numpy as jnp
from jax import lax
from jax.experimental import pallas as pl
from jax.experimental.pallas import tpu as pltpu
from jax.experimental.pallas import tpu_sc as plsc

_LANES = 16
_HALF = 100


def _make_pool(B, S, D, nc, ns):
  NW = nc * ns
  BPW = B // NW
  assert S == 2 * _HALF and D % _LANES == 0 and B % NW == 0
  nchunks = D // _LANES
  scale = 1.0 / S
  mesh = plsc.VectorSubcoreMesh(core_axis_name="c", subcore_axis_name="s")

  def body(x_hbm, table_hbm, out_hbm, idx_v, rows_v, out_v, sem0, sem1):
    wid = lax.axis_index("s") * nc + lax.axis_index("c")
    base = wid * BPW
    pltpu.sync_copy(x_hbm.at[pl.ds(base * 2, BPW * 2)], idx_v)
    sems = (sem0, sem1)

    def start(i, buf, sem):
      pltpu.async_copy(table_hbm.at[idx_v.at[2 * i]],
                       rows_v.at[buf, pl.ds(0, _HALF)], sem)
      pltpu.async_copy(table_hbm.at[idx_v.at[2 * i + 1]],
                       rows_v.at[buf, pl.ds(_HALF, _HALF)], sem)

    def wait(i, buf, sem):
      pltpu.make_async_copy(table_hbm.at[idx_v.at[2 * i]],
                            rows_v.at[buf, pl.ds(0, _HALF)], sem).wait()
      pltpu.make_async_copy(table_hbm.at[idx_v.at[2 * i + 1]],
                            rows_v.at[buf, pl.ds(_HALF, _HALF)], sem).wait()

    start(0, 0, sem0)

    def outer(i2, carry):
      for b in (0, 1):
        i = i2 * 2 + b
        nxt = i + 1

        @pl.when(nxt < BPW)
        def _():
          start(nxt, 1 - b, sems[1 - b])

        wait(i, b, sems[b])

        def rbody(r, acc):
          return tuple(acc[c] + rows_v[b, r, pl.ds(c * _LANES, _LANES)]
                       for c in range(nchunks))

        zero = jnp.zeros((_LANES,), jnp.float32)
        acc = lax.fori_loop(0, S, rbody, (zero,) * nchunks)
        for c in range(nchunks):
          out_v[i, pl.ds(c * _LANES, _LANES)] = acc[c] * scale
      return carry

    lax.fori_loop(0, BPW // 2, outer, 0)
    pltpu.sync_copy(out_v, out_hbm.at[pl.ds(base, BPW)])

  return pl.kernel(
      body,
      out_type=jax.ShapeDtypeStruct((B, D), jnp.float32),
      mesh=mesh,
      scratch_types=[
          pltpu.VMEM((2 * BPW, _HALF), jnp.int32),
          pltpu.VMEM((2, S, D), jnp.float32),
          pltpu.VMEM((BPW, D), jnp.float32),
          pltpu.SemaphoreType.DMA,
          pltpu.SemaphoreType.DMA,
      ],
      compiler_params=pltpu.CompilerParams(use_tc_tiling_on_sc=False),
  )


def _make_transpose(V, D, nc, ns):
  NW = nc * ns
  NT = V // 128
  per_w = NT // NW
  extra = NT % NW
  assert D == 64
  mesh = plsc.VectorSubcoreMesh(core_axis_name="c", subcore_axis_name="s")

  def body(t_hbm, c_hbm, vin, vout, semi0, semi1):
    wid = lax.axis_index("s") * nc + lax.axis_index("c")
    start = wid * per_w + jnp.minimum(wid, extra)
    cnt = per_w + jnp.where(wid < extra, 1, 0)
    sems = (semi0, semi1)

    lanes = lax.iota(jnp.int32, 16)
    rowpats = [(lanes >> 1) + 8 * w for w in range(8)]
    colpat = (lanes & 1) * D

    def start_in(j, b):
      col = pl.multiple_of(j * 128, 128)
      pltpu.async_copy(t_hbm.at[:, pl.ds(col, 128)], vin.at[b], sems[b])

    def wait_in(j, b):
      col = pl.multiple_of(j * 128, 128)
      pltpu.make_async_copy(t_hbm.at[:, pl.ds(col, 128)], vin.at[b],
                            sems[b]).wait()

    @pl.when(cnt > 0)
    def _():
      start_in(start, 0)

    def outer(i2, carry):
      for b in (0, 1):
        i = i2 * 2 + b

        @pl.when(i < cnt)
        def _():
          @pl.when(i + 1 < cnt)
          def _():
            start_in(start + i + 1, 1 - b)

          wait_in(start + i, b)

          def dbody(d, c):
            colv = colpat + d
            for w in range(8):
              plsc.store_scatter(vout, [rowpats[w], colv],
                                 vin[b, d, pl.ds(16 * w, 16)])
            return c

          lax.fori_loop(0, D, dbody, 0)
          pltpu.sync_copy(vout, c_hbm.at[pl.ds((start + i) * 64, 64)])
      return carry

    lax.fori_loop(0, (per_w + 2) // 2, outer, 0)

  return pl.kernel(
      body,
      out_type=jax.ShapeDtypeStruct((V // 2, 2 * D), jnp.float32),
      mesh=mesh,
      scratch_types=[
          pltpu.VMEM((2, D, 128), jnp.float32),
          pltpu.VMEM((64, 128), jnp.float32),
          pltpu.SemaphoreType.DMA,
          pltpu.SemaphoreType.DMA,
      ],
      compiler_params=pltpu.CompilerParams(needs_layout_passes=False),
  )


def _mlp_body(p_ref, w1_ref, b1_ref, w2_ref, b2_ref, o_ref):
  h = jnp.dot(p_ref[...], w1_ref[...], preferred_element_type=jnp.float32)
  h = jnp.maximum(h + b1_ref[...], 0.0)
  logits = jnp.dot(h, w2_ref[...], preferred_element_type=jnp.float32)
  logits = logits + b2_ref[...]
  m = jnp.max(logits, axis=1, keepdims=True)
  e = jnp.exp(logits - m)
  o_ref[...] = e / jnp.sum(e, axis=1, keepdims=True)


def kernel(x, table, W1, b1, W2, b2):
  B, S = x.shape
  V, D = table.shape
  H = W1.shape[1]
  C = W2.shape[1]

  info = plsc.get_sparse_core_info()
  nc, ns = info.num_cores, info.num_subcores

  trans = _make_transpose(V, D, nc, ns)
  tab2 = trans(table.T)
  rem = V % 128
  tail = table[V - rem:].reshape(rem // 2, 2 * D)
  tab2 = lax.dynamic_update_slice(tab2, tail, ((V - rem) // 2, 0))
  Z = tab2.reshape(V, D)

  pool = _make_pool(B, S, D, nc, ns)
  pooled = pool(x.reshape(B * 2, S // 2), Z)

  CP = 128
  W2p = jnp.zeros((H, CP), jnp.float32).at[:, :C].set(W2)
  b2p = jnp.full((1, CP), -1e30, jnp.float32).at[0, :C].set(b2)
  BLK = 1024
  out = pl.pallas_call(
      _mlp_body,
      out_shape=jax.ShapeDtypeStruct((B, CP), jnp.float32),
      grid=(B // BLK,),
      in_specs=[
          pl.BlockSpec((BLK, D), lambda i: (i, 0)),
          pl.BlockSpec((D, H), lambda i: (0, 0)),
          pl.BlockSpec((1, H), lambda i: (0, 0)),
          pl.BlockSpec((H, CP), lambda i: (0, 0)),
          pl.BlockSpec((1, CP), lambda i: (0, 0)),
      ],
      out_specs=pl.BlockSpec((BLK, CP), lambda i: (i, 0)),
  )(pooled, W1, b1.reshape(1, H), W2p, b2p)
  return out[:, :C]


# speedup vs baseline: 4.4037x; 4.4037x over previous
import functools

import jax
import jax.numpy as jnp
from jax import lax
from jax.experimental import pallas as pl
from jax.experimental.pallas import tpu as pltpu
from jax.experimental.pallas import tpu_sc as plsc

_LANES = 16
_HALF = 100


def _make_pool(B, S, D, nc, ns):
  NW = nc * ns
  BPW = B // NW
  assert S == 2 * _HALF and D % _LANES == 0 and B % NW == 0
  nchunks = D // _LANES
  scale = 1.0 / S
  mesh = plsc.VectorSubcoreMesh(core_axis_name="c", subcore_axis_name="s")

  def body(x_hbm, table_hbm, out_hbm, idx_v, rows_v, out_v, sem0, sem1):
    wid = lax.axis_index("s") * nc + lax.axis_index("c")
    base = wid * BPW
    pltpu.sync_copy(x_hbm.at[pl.ds(base * 2, BPW * 2)], idx_v)
    sems = (sem0, sem1)

    def start(i, buf, sem):
      pltpu.async_copy(table_hbm.at[idx_v.at[2 * i]],
                       rows_v.at[buf, pl.ds(0, _HALF)], sem)
      pltpu.async_copy(table_hbm.at[idx_v.at[2 * i + 1]],
                       rows_v.at[buf, pl.ds(_HALF, _HALF)], sem)

    def wait(i, buf, sem):
      pltpu.make_async_copy(table_hbm.at[idx_v.at[2 * i]],
                            rows_v.at[buf, pl.ds(0, _HALF)], sem).wait()
      pltpu.make_async_copy(table_hbm.at[idx_v.at[2 * i + 1]],
                            rows_v.at[buf, pl.ds(_HALF, _HALF)], sem).wait()

    start(0, 0, sem0)

    def outer(i2, carry):
      for b in (0, 1):
        i = i2 * 2 + b
        nxt = i + 1

        @pl.when(nxt < BPW)
        def _():
          start(nxt, 1 - b, sems[1 - b])

        wait(i, b, sems[b])

        def rbody(r4, acc):
          out = list(acc)
          for u in range(4):
            r = r4 * 4 + u
            vals = [rows_v[b, r, pl.ds(c * _LANES, _LANES)]
                    for c in range(nchunks)]
            out = [out[c] + vals[c] for c in range(nchunks)]
          return tuple(out)

        zero = jnp.zeros((_LANES,), jnp.float32)
        acc = lax.fori_loop(0, S // 4, rbody, (zero,) * nchunks)
        for c in range(nchunks):
          out_v[i, pl.ds(c * _LANES, _LANES)] = acc[c] * scale
      return carry

    lax.fori_loop(0, BPW // 2, outer, 0)
    pltpu.sync_copy(out_v, out_hbm.at[pl.ds(base, BPW)])

  return pl.kernel(
      body,
      out_type=jax.ShapeDtypeStruct((B, D), jnp.float32),
      mesh=mesh,
      scratch_types=[
          pltpu.VMEM((2 * BPW, _HALF), jnp.int32),
          pltpu.VMEM((2, S, D), jnp.float32),
          pltpu.VMEM((BPW, D), jnp.float32),
          pltpu.SemaphoreType.DMA,
          pltpu.SemaphoreType.DMA,
      ],
      compiler_params=pltpu.CompilerParams(use_tc_tiling_on_sc=False),
  )


def _make_transpose(V, D, nc, ns):
  NW = nc * ns
  CW = 384
  NP = V // CW
  per_w = NP // NW
  extra = NP % NW
  L = 16
  assert D == 64
  mesh = plsc.VectorSubcoreMesh(core_axis_name="c", subcore_axis_name="s")

  def body(t_hbm, c_hbm, vin, vout, semi0, semi1, semo0, semo1):
    wid = lax.axis_index("s") * nc + lax.axis_index("c")
    start = wid * per_w + jnp.minimum(wid, extra)
    cnt = per_w + jnp.where(wid < extra, 1, 0)
    semi = (semi0, semi1)
    semo = (semo0, semo1)
    lanes = lax.iota(jnp.int32, L)
    vpats = [((lanes + k) & (L - 1)) for k in range(L)]
    bconsts = [lanes * 0, (lanes * 0) + 1]

    def start_in(p, b):
      col = pl.multiple_of(p * CW, 128)
      pltpu.async_copy(t_hbm.at[:, pl.ds(col, CW)], vin.at[b], semi[b])

    def wait_in(p, b):
      col = pl.multiple_of(p * CW, 128)
      pltpu.make_async_copy(t_hbm.at[:, pl.ds(col, CW)], vin.at[b],
                            semi[b]).wait()

    def start_out(p, b):
      r0 = pl.multiple_of(p * (CW // 2), 8)
      pltpu.async_copy(vout.at[b], c_hbm.at[pl.ds(r0, CW // 2)], semo[b])

    def wait_out(b):
      pltpu.make_async_copy(vout.at[b], c_hbm.at[pl.ds(0, CW // 2)],
                            semo[b]).wait()

    def scatter_pair(b):
      bc = bconsts[b]

      def vblock(vb, carry):
        v0 = vb * L
        for d0 in (0, 16, 32, 48):
          dvec = lanes + d0
          for kc in (0, 8):
            vvs, vals = [], []
            for k in range(kc, kc + 8):
              vv = vpats[k] + v0
              vvs.append(vv)
              vals.append(plsc.load_gather(vin, [bc, dvec, vv]))
            for q in range(8):
              tv = vvs[q] >> 1
              cv = ((vvs[q] & 1) << 6) + dvec
              plsc.store_scatter(vout, [bc, tv, cv], vals[q])
        return carry

      lax.fori_loop(0, CW // L, vblock, 0)

    @pl.when(cnt > 0)
    def _():
      start_in(start, 0)

    def outer(i2, carry):
      for b in (0, 1):
        i = i2 * 2 + b

        @pl.when(i < cnt)
        def _():
          @pl.when(i + 1 < cnt)
          def _():
            start_in(start + i + 1, 1 - b)

          wait_in(start + i, b)

          @pl.when(i >= 2)
          def _():
            wait_out(b)

          scatter_pair(b)
          start_out(start + i, b)
      return carry

    lax.fori_loop(0, (per_w + 2) // 2, outer, 0)
    wait_out(0)
    @pl.when(cnt > 1)
    def _():
      wait_out(1)

  return pl.kernel(
      body,
      out_type=jax.ShapeDtypeStruct((V // 2, 2 * D), jnp.float32),
      mesh=mesh,
      scratch_types=[
          pltpu.VMEM((2, D, CW), jnp.float32),
          pltpu.VMEM((2, CW // 2, 128), jnp.float32),
          pltpu.SemaphoreType.DMA,
          pltpu.SemaphoreType.DMA,
          pltpu.SemaphoreType.DMA,
          pltpu.SemaphoreType.DMA,
      ],
      compiler_params=pltpu.CompilerParams(needs_layout_passes=False),
  )


def _mlp_body(p_ref, w1_ref, b1_ref, w2_ref, b2_ref, o_ref):
  h = jnp.dot(p_ref[...], w1_ref[...], preferred_element_type=jnp.float32)
  h = jnp.maximum(h + b1_ref[...], 0.0)
  logits = jnp.dot(h, w2_ref[...], preferred_element_type=jnp.float32)
  logits = logits + b2_ref[...]
  m = jnp.max(logits, axis=1, keepdims=True)
  e = jnp.exp(logits - m)
  o_ref[...] = e / jnp.sum(e, axis=1, keepdims=True)


def kernel(x, table, W1, b1, W2, b2):
  B, S = x.shape
  V, D = table.shape
  H = W1.shape[1]
  C = W2.shape[1]

  info = plsc.get_sparse_core_info()
  nc, ns = info.num_cores, info.num_subcores

  trans = _make_transpose(V, D, nc, ns)
  tab2 = trans(table.T)
  rem = V % 128
  tail = table[V - rem:].reshape(rem // 2, 2 * D)
  tab2 = lax.dynamic_update_slice(tab2, tail, ((V - rem) // 2, 0))
  Z = tab2.reshape(V, D)

  pool = _make_pool(B, S, D, nc, ns)
  pooled = pool(x.reshape(B * 2, S // 2), Z)

  CP = 128
  W2p = jnp.zeros((H, CP), jnp.float32).at[:, :C].set(W2)
  b2p = jnp.full((1, CP), -1e30, jnp.float32).at[0, :C].set(b2)
  BLK = 1024
  out = pl.pallas_call(
      _mlp_body,
      out_shape=jax.ShapeDtypeStruct((B, CP), jnp.float32),
      grid=(B // BLK,),
      in_specs=[
          pl.BlockSpec((BLK, D), lambda i: (i, 0)),
          pl.BlockSpec((D, H), lambda i: (0, 0)),
          pl.BlockSpec((1, H), lambda i: (0, 0)),
          pl.BlockSpec((H, CP), lambda i: (0, 0)),
          pl.BlockSpec((1, CP), lambda i: (0, 0)),
      ],
      out_specs=pl.BlockSpec((BLK, CP), lambda i: (i, 0)),
  )(pooled, W1, b1.reshape(1, H), W2p, b2p)
  return out[:, :C]
